# baseline (speedup 1.0000x reference)
import jax
import jax.numpy as jnp
from jax import lax
from jax.experimental import pallas as pl
from jax.experimental.pallas import tpu as pltpu
from jax.experimental.pallas import tpu_sc as plsc

F32 = jnp.float32
I32 = jnp.int32
L = 16
N = 1000
NPAD = 1024
NCH1 = (N + L - 1) // L
M = 64
B = 8
CONF_THRESH = 0.7
IOU_THRESH = 0.5
NEG = -1e30


def _recip(x):
    r = plsc.bitcast(jnp.full((L,), 0x7EF311C3, I32) - plsc.bitcast(x, I32),
                     F32)
    for _ in range(3):
        r = r * (2.0 - x * r)
    return r


def _body(pred_hbm, gt_hbm, sz_hbm, out_hbm, pred_v, gt_v, sz_v, cconf, cpx2,
          cpy2, cpar, gmask, cconf2, cpx22, cpy22, cpar2, gtc, res_v,
          redbuf, ap1, ap2, shared):
    c = lax.axis_index("c")
    s = lax.axis_index("s")
    iota = lax.iota(I32, L)
    active = (c == 0) & (s < B)

    @pl.when(active)
    def _work():
        b = s
        pltpu.sync_copy(pred_hbm, pred_v)
        pltpu.sync_copy(gt_hbm, gt_v)
        pltpu.sync_copy(sz_hbm, sz_v)
        pbase = jnp.broadcast_to(b * (N * 6), (L,))
        gbase = jnp.broadcast_to(b * (M * 5), (L,))
        wv = plsc.load_gather(sz_v, [jnp.broadcast_to(2 * b, (L,))])
        hv = plsc.load_gather(sz_v, [jnp.broadcast_to(2 * b + 1, (L,))])
        zero = jnp.zeros((L,), F32)

        for k in range(M // L):
            gtc[pl.ds(3 * M + k * L, L)] = zero
        csc = jnp.zeros((L,), I32)
        for k in range(M // L):
            gidx = gbase + (iota + k * L) * 5
            gcls = plsc.load_gather(gt_v, [gidx])
            gx = plsc.load_gather(gt_v, [gidx + 1]) * wv
            gy = plsc.load_gather(gt_v, [gidx + 2]) * hv
            gw = plsc.load_gather(gt_v, [gidx + 3]) * wv
            gh = plsc.load_gather(gt_v, [gidx + 4]) * hv
            gx1 = gx - gw * 0.5
            gy1 = gy - gh * 0.5
            gx2 = gx1 + gw
            gy2 = gy1 + gh
            area = (gx2 - gx1) * (gy2 - gy1)
            gvb = gcls == 0.0
            cs = plsc.cumsum(jnp.where(gvb, 1, 0).astype(I32)) + csc
            csc = jnp.broadcast_to(jnp.max(cs), (L,))
            pos = cs - 1
            plsc.store_scatter(gtc, [pos], gx1, mask=gvb)
            plsc.store_scatter(gtc, [pos + M], gy1, mask=gvb)
            plsc.store_scatter(gtc, [pos + 2 * M], area, mask=gvb)
        ngv = jnp.max(csc)
        ngv_v = jnp.broadcast_to(ngv, (L,))

        def compb(j, cnt_v):
            pidx = pbase + jnp.minimum(iota + j * L, N - 1) * 6
            pcls = plsc.load_gather(pred_v, [pidx])
            pconf = plsc.load_gather(pred_v, [pidx + 1])
            px1 = plsc.load_gather(pred_v, [pidx + 2])
            py1 = plsc.load_gather(pred_v, [pidx + 3])
            px2 = plsc.load_gather(pred_v, [pidx + 4])
            py2 = plsc.load_gather(pred_v, [pidx + 5])
            pvb = ((pcls == 0.0) & (pconf >= CONF_THRESH) &
                   ((iota + j * L) < N))
            inc = plsc.cumsum(jnp.where(pvb, 1, 0).astype(I32))
            pos = inc + (cnt_v - 1)
            plsc.store_scatter(cconf, [pos], pconf, mask=pvb)
            plsc.store_scatter(cpx2, [pos], px2, mask=pvb)
            plsc.store_scatter(cpy2, [pos], py2, mask=pvb)
            plsc.store_scatter(cpar, [pos], (px2 - px1) * (py2 - py1),
                               mask=pvb)
            return cnt_v + plsc.all_reduce_population_count(pvb)

        cnt_v = lax.fori_loop(0, NCH1, compb, jnp.zeros((L,), I32))
        n_p = jnp.max(cnt_v)
        nch = lax.shift_right_logical(n_p + (L - 1), 4)

        def zb(j, carry):
            gmask[pl.ds(j * L, L)] = zero
            cconf2[pl.ds(j * L, L)] = jnp.full((L,), -1.0, F32)
            return carry

        lax.fori_loop(0, nch, zb, 0)

        def pfg(g, carry):
            gvec = jnp.broadcast_to(g, (L,))
            gx1 = plsc.load_gather(gtc, [gvec])
            gy1 = plsc.load_gather(gtc, [gvec + M])
            ga = plsc.load_gather(gtc, [gvec + 2 * M])

            def pfj(j, carry2):
                sl = pl.ds(j * L, L)
                px2 = cpx2[sl]
                py2 = cpy2[sl]
                pa = cpar[sl]
                ov = jnp.abs(px2 - gx1) * jnp.abs(py2 - gy1)
                mina = jnp.minimum(ga, pa)
                ov = jnp.where(ov > mina, 0.0, ov)
                un = pa + ga - ov
                ps = ov > IOU_THRESH * un
                gm = gmask[sl]
                gmask[sl] = jnp.where(ps, 1.0, gm)
                return carry2

            lax.fori_loop(0, nch, pfj, 0)
            return carry

        lax.fori_loop(0, ngv, pfg, 0)

        def comp2(j, cnt2_v):
            sl = pl.ds(j * L, L)
            keep = gmask[sl] > 0.5
            inc = plsc.cumsum(jnp.where(keep, 1, 0).astype(I32))
            pos = inc + (cnt2_v - 1)
            plsc.store_scatter(cconf2, [pos], cconf[sl], mask=keep)
            plsc.store_scatter(cpx22, [pos], cpx2[sl], mask=keep)
            plsc.store_scatter(cpy22, [pos], cpy2[sl], mask=keep)
            plsc.store_scatter(cpar2, [pos], cpar[sl], mask=keep)
            return cnt2_v + plsc.all_reduce_population_count(keep)

        cnt2_v = lax.fori_loop(0, nch, comp2, jnp.zeros((L,), I32))
        n2 = jnp.max(cnt2_v)
        nch2 = lax.shift_right_logical(n2 + (L - 1), 4)

        one16 = jnp.full((L,), 1.0, F32)
        negv = jnp.full((L,), -1.0, F32)

        def step(si, carry):
            def selA(j, mv):
                return jnp.maximum(mv, cconf2[pl.ds(j * L, L)])

            mv = lax.fori_loop(0, nch2, selA, negv)
            gsel = jnp.max(mv)

            def selB(j, st):
                j_v, found = st
                hit = plsc.all_reduce_population_count(
                    cconf2[pl.ds(j * L, L)] == gsel)
                newj = jnp.where((hit > 0) & (found == 0),
                                 jnp.broadcast_to(j, (L,)), j_v)
                return (newj, jnp.where(hit > 0, 1, found))

            j_v, _ = lax.fori_loop(0, nch2, selB,
                                   (jnp.zeros((L,), I32),
                                    jnp.zeros((L,), I32)))
            v_j = plsc.load_gather(cconf2, [j_v * L + iota])
            lane_v = plsc.all_reduce_ffs(v_j == gsel)
            bpos_v = j_v * L + lane_v
            plsc.store_scatter(cconf2, [bpos_v], negv, mask=iota == 0)
            px2 = plsc.load_gather(cpx22, [bpos_v])
            py2 = plsc.load_gather(cpy22, [bpos_v])
            p_area = plsc.load_gather(cpar2, [bpos_v])
            mious = []
            for k in range(M // L):
                gx1 = gtc[pl.ds(0 * M + k * L, L)]
                gy1 = gtc[pl.ds(1 * M + k * L, L)]
                area = gtc[pl.ds(2 * M + k * L, L)]
                mt = gtc[pl.ds(3 * M + k * L, L)]
                ov = jnp.abs(px2 - gx1) * jnp.abs(py2 - gy1)
                mina = jnp.minimum(area, p_area)
                ov = jnp.where(ov > mina, 0.0, ov)
                un = p_area + area - ov
                un = jnp.where(un == 0.0, 1e-12, un)
                iou = ov * _recip(un)
                passed = ((mt < 0.5) & (ov > IOU_THRESH * un) &
                          ((iota + k * L) < ngv_v))
                mious.append(jnp.where(passed, iou, NEG))
            mall = jnp.maximum(jnp.maximum(mious[0], mious[1]),
                               jnp.maximum(mious[2], mious[3]))
            gmax = jnp.max(mall)
            any_pass = gmax > 0.25
            found = jnp.zeros((L,), I32)
            rank_v = jnp.zeros((L,), I32)
            for k in range(M // L):
                hk = mious[k] == gmax
                lane_k = plsc.all_reduce_ffs(hk)
                has_k = (plsc.all_reduce_population_count(hk) > 0) & \
                        (found == 0)
                passed_c = mious[k] > 0.25
                cnt_here = plsc.all_reduce_population_count(
                    passed_c & (iota < lane_k))
                cnt_full = plsc.all_reduce_population_count(passed_c)
                rank_v = rank_v + jnp.where(found > 0, 0,
                                            jnp.where(has_k, cnt_here,
                                                      cnt_full))
                found = jnp.where(has_k, 1, found)
            plsc.store_scatter(gtc, [rank_v + 3 * M], one16,
                               mask=(iota == 0) & any_pass)
            return carry

        lax.fori_loop(0, n2, step, 0)

        tpa = jnp.zeros((L,), F32)
        for k in range(M // L):
            tpa = tpa + gtc[pl.ds(3 * M + k * L, L)]
        tp = jnp.broadcast_to(jnp.sum(tpa), (L,))
        g = ngv_v.astype(F32)
        npf = jnp.broadcast_to(n_p.astype(F32), (L,))
        has = n_p > 0
        prec = jnp.where(has, tp * _recip(jnp.maximum(npf, 1.0)), 0.0)
        rec = jnp.where(has, tp * _recip(jnp.maximum(g, 1.0)), 0.0)
        res_v[pl.ds(0, L)] = jnp.where(iota == b, prec, 0.0)
        pltpu.sync_copy(res_v, shared.at[pl.ds(b * L, L)])
        res_v[pl.ds(0, L)] = jnp.where(iota == b, rec, 0.0)
        pltpu.sync_copy(res_v, shared.at[pl.ds((B + b) * L, L)])

    plsc.subcore_barrier()

    @pl.when((c == 0) & (s == 0))
    def _reduce():
        pltpu.sync_copy(shared, redbuf)
        prec_vec = jnp.zeros((L,), F32)
        rec_vec = jnp.zeros((L,), F32)
        for i in range(B):
            prec_vec = prec_vec + redbuf[pl.ds(i * L, L)]
            rec_vec = rec_vec + redbuf[pl.ds((B + i) * L, L)]
        zero = jnp.zeros((L,), F32)
        ap1[pl.ds(0, L)] = zero
        ap1[pl.ds(L, L)] = zero
        ap2[pl.ds(0, L)] = zero
        ap2[pl.ds(L, L)] = zero
        plsc.store_scatter(ap1, [iota + 1], rec_vec, mask=iota < B)
        plsc.store_scatter(ap1, [jnp.full((L,), B + 1, I32)],
                           jnp.full((L,), 1.0, F32), mask=iota == 0)
        plsc.store_scatter(ap2, [iota + 1], prec_vec, mask=iota < B)
        mp = ap2[pl.ds(0, L)]
        mp = lax.rev(plsc.cummax(lax.rev(mp, (0,))), (0,))
        ap2[pl.ds(0, L)] = mp
        mrec = ap1[pl.ds(0, L)]
        mrec_n = plsc.load_gather(ap1, [iota + 1])
        mpre_n = plsc.load_gather(ap2, [iota + 1])
        terms = jnp.where(iota < B + 1, (mrec_n - mrec) * mpre_n, 0.0)
        apv = jnp.sum(terms)
        res_v[pl.ds(0, L)] = zero + apv
        pltpu.sync_copy(res_v, out_hbm)


def kernel(predicts, ground_truths, image_sizes):
    pred_arr = predicts.astype(F32).reshape(B * N * 6)
    gt_arr = ground_truths.astype(F32).reshape(B * M * 5)
    sz_arr = image_sizes.astype(F32).reshape(2 * B)

    mesh = plsc.VectorSubcoreMesh(core_axis_name="c", subcore_axis_name="s")
    out = pl.kernel(
        _body,
        out_type=jax.ShapeDtypeStruct((L,), F32),
        mesh=mesh,
        compiler_params=pltpu.CompilerParams(needs_layout_passes=False),
        scratch_types=[
            pltpu.VMEM((B * N * 6,), F32),
            pltpu.VMEM((B * M * 5,), F32),
            pltpu.VMEM((2 * B,), F32),
            pltpu.VMEM((NPAD,), F32),
            pltpu.VMEM((NPAD,), F32),
            pltpu.VMEM((NPAD,), F32),
            pltpu.VMEM((NPAD,), F32),
            pltpu.VMEM((NPAD,), F32),
            pltpu.VMEM((NPAD,), F32),
            pltpu.VMEM((NPAD,), F32),
            pltpu.VMEM((NPAD,), F32),
            pltpu.VMEM((NPAD,), F32),
            pltpu.VMEM((4 * M,), F32),
            pltpu.VMEM((L,), F32),
            pltpu.VMEM((2 * B * L,), F32),
            pltpu.VMEM((2 * L,), F32),
            pltpu.VMEM((2 * L,), F32),
            pltpu.VMEM_SHARED((2 * B * L,), F32),
        ],
    )(pred_arr, gt_arr, sz_arr)
    return out[0]


# speedup vs baseline: 1.2274x; 1.2274x over previous
import jax
import jax.numpy as jnp
from jax import lax
from jax.experimental import pallas as pl
from jax.experimental.pallas import tpu as pltpu
from jax.experimental.pallas import tpu_sc as plsc

F32 = jnp.float32
I32 = jnp.int32
L = 16
NPAD = 1024
NCHUNKS = NPAD // L
M = 64
B = 8
CONF_THRESH = 0.7
IOU_THRESH = 0.5
NEG = -1e30


def _recip(x):
    r = plsc.bitcast(jnp.full((L,), 0x7EF311C3, I32) - plsc.bitcast(x, I32),
                     F32)
    for _ in range(3):
        r = r * (2.0 - x * r)
    return r


def _body(pred_hbm, gt_hbm, out_hbm, pred_v, gt_v, cconf, cpx2, cpy2, cpar,
          gmask, cconf2, cpx22, cpy22, cpar2, gtc, res_v, redbuf, ap1, ap2,
          shared):
    c = lax.axis_index("c")
    s = lax.axis_index("s")
    iota = lax.iota(I32, L)
    active = (c == 0) & (s < B)

    @pl.when(active)
    def _work():
        b = s
        pltpu.sync_copy(pred_hbm.at[b], pred_v)
        pltpu.sync_copy(gt_hbm.at[b], gt_v)
        wv = plsc.load_gather(gt_v, [jnp.full((L,), 5 * M, I32)])
        hv = plsc.load_gather(gt_v, [jnp.full((L,), 5 * M + 1, I32)])
        zero = jnp.zeros((L,), F32)

        for k in range(M // L):
            gtc[pl.ds(3 * M + k * L, L)] = zero
        csc = jnp.zeros((L,), I32)
        for k in range(M // L):
            gcls = gt_v[pl.ds(0 * M + k * L, L)]
            gx = gt_v[pl.ds(1 * M + k * L, L)] * wv
            gy = gt_v[pl.ds(2 * M + k * L, L)] * hv
            gw = gt_v[pl.ds(3 * M + k * L, L)] * wv
            gh = gt_v[pl.ds(4 * M + k * L, L)] * hv
            gx1 = gx - gw * 0.5
            gy1 = gy - gh * 0.5
            gx2 = gx1 + gw
            gy2 = gy1 + gh
            area = (gx2 - gx1) * (gy2 - gy1)
            gvb = gcls == 0.0
            cs = plsc.cumsum(jnp.where(gvb, 1, 0).astype(I32)) + csc
            csc = jnp.broadcast_to(jnp.max(cs), (L,))
            pos = cs - 1
            plsc.store_scatter(gtc, [pos], gx1, mask=gvb)
            plsc.store_scatter(gtc, [pos + M], gy1, mask=gvb)
            plsc.store_scatter(gtc, [pos + 2 * M], area, mask=gvb)
        ngv = jnp.max(csc)
        ngv_v = jnp.broadcast_to(ngv, (L,))

        def compb(j, cnt_v):
            base = j * L
            pcls = pred_v[pl.ds(base, L)]
            pconf = pred_v[pl.ds(NPAD + base, L)]
            px1 = pred_v[pl.ds(2 * NPAD + base, L)]
            py1 = pred_v[pl.ds(3 * NPAD + base, L)]
            px2 = pred_v[pl.ds(4 * NPAD + base, L)]
            py2 = pred_v[pl.ds(5 * NPAD + base, L)]
            pvb = (pcls == 0.0) & (pconf >= CONF_THRESH)
            inc = plsc.cumsum(jnp.where(pvb, 1, 0).astype(I32))
            pos = inc + (cnt_v - 1)
            plsc.store_scatter(cconf, [pos], pconf, mask=pvb)
            plsc.store_scatter(cpx2, [pos], px2, mask=pvb)
            plsc.store_scatter(cpy2, [pos], py2, mask=pvb)
            plsc.store_scatter(cpar, [pos], (px2 - px1) * (py2 - py1),
                               mask=pvb)
            return cnt_v + plsc.all_reduce_population_count(pvb)

        cnt_v = lax.fori_loop(0, NCHUNKS, compb, jnp.zeros((L,), I32))
        n_p = jnp.max(cnt_v)
        nch = lax.shift_right_logical(n_p + (L - 1), 4)

        def zb(j, carry):
            gmask[pl.ds(j * L, L)] = zero
            cconf2[pl.ds(j * L, L)] = jnp.full((L,), -1.0, F32)
            return carry

        lax.fori_loop(0, nch, zb, 0)

        def pfg(g, carry):
            gvec = jnp.broadcast_to(g, (L,))
            gx1 = plsc.load_gather(gtc, [gvec])
            gy1 = plsc.load_gather(gtc, [gvec + M])
            ga = plsc.load_gather(gtc, [gvec + 2 * M])

            def pfj(j, carry2):
                sl = pl.ds(j * L, L)
                px2 = cpx2[sl]
                py2 = cpy2[sl]
                pa = cpar[sl]
                ov = jnp.abs(px2 - gx1) * jnp.abs(py2 - gy1)
                mina = jnp.minimum(ga, pa)
                ov = jnp.where(ov > mina, 0.0, ov)
                un = pa + ga - ov
                ps = ov > IOU_THRESH * un
                gm = gmask[sl]
                gmask[sl] = jnp.where(ps, 1.0, gm)
                return carry2

            lax.fori_loop(0, nch, pfj, 0)
            return carry

        lax.fori_loop(0, ngv, pfg, 0)

        def comp2(j, cnt2_v):
            sl = pl.ds(j * L, L)
            keep = gmask[sl] > 0.5
            inc = plsc.cumsum(jnp.where(keep, 1, 0).astype(I32))
            pos = inc + (cnt2_v - 1)
            plsc.store_scatter(cconf2, [pos], cconf[sl], mask=keep)
            plsc.store_scatter(cpx22, [pos], cpx2[sl], mask=keep)
            plsc.store_scatter(cpy22, [pos], cpy2[sl], mask=keep)
            plsc.store_scatter(cpar2, [pos], cpar[sl], mask=keep)
            return cnt2_v + plsc.all_reduce_population_count(keep)

        cnt2_v = lax.fori_loop(0, nch, comp2, jnp.zeros((L,), I32))
        n2 = jnp.max(cnt2_v)
        nch2 = lax.shift_right_logical(n2 + (L - 1), 4)

        one16 = jnp.full((L,), 1.0, F32)
        negv = jnp.full((L,), -1.0, F32)

        def step(si, carry):
            def selA(j, mv):
                return jnp.maximum(mv, cconf2[pl.ds(j * L, L)])

            mv = lax.fori_loop(0, nch2, selA, negv)
            gsel = jnp.max(mv)

            def selB(j, st):
                j_v, found = st
                hit = plsc.all_reduce_population_count(
                    cconf2[pl.ds(j * L, L)] == gsel)
                newj = jnp.where((hit > 0) & (found == 0),
                                 jnp.broadcast_to(j, (L,)), j_v)
                return (newj, jnp.where(hit > 0, 1, found))

            j_v, _ = lax.fori_loop(0, nch2, selB,
                                   (jnp.zeros((L,), I32),
                                    jnp.zeros((L,), I32)))
            v_j = plsc.load_gather(cconf2, [j_v * L + iota])
            lane_v = plsc.all_reduce_ffs(v_j == gsel)
            bpos_v = j_v * L + lane_v
            plsc.store_scatter(cconf2, [bpos_v], negv, mask=iota == 0)
            px2 = plsc.load_gather(cpx22, [bpos_v])
            py2 = plsc.load_gather(cpy22, [bpos_v])
            p_area = plsc.load_gather(cpar2, [bpos_v])
            mious = []
            for k in range(M // L):
                gx1 = gtc[pl.ds(0 * M + k * L, L)]
                gy1 = gtc[pl.ds(1 * M + k * L, L)]
                area = gtc[pl.ds(2 * M + k * L, L)]
                mt = gtc[pl.ds(3 * M + k * L, L)]
                ov = jnp.abs(px2 - gx1) * jnp.abs(py2 - gy1)
                mina = jnp.minimum(area, p_area)
                ov = jnp.where(ov > mina, 0.0, ov)
                un = p_area + area - ov
                un = jnp.where(un == 0.0, 1e-12, un)
                iou = ov * _recip(un)
                passed = ((mt < 0.5) & (ov > IOU_THRESH * un) &
                          ((iota + k * L) < ngv_v))
                mious.append(jnp.where(passed, iou, NEG))
            mall = jnp.maximum(jnp.maximum(mious[0], mious[1]),
                               jnp.maximum(mious[2], mious[3]))
            gmax = jnp.max(mall)
            any_pass = gmax > 0.25
            found = jnp.zeros((L,), I32)
            rank_v = jnp.zeros((L,), I32)
            for k in range(M // L):
                hk = mious[k] == gmax
                lane_k = plsc.all_reduce_ffs(hk)
                has_k = (plsc.all_reduce_population_count(hk) > 0) & \
                        (found == 0)
                passed_c = mious[k] > 0.25
                cnt_here = plsc.all_reduce_population_count(
                    passed_c & (iota < lane_k))
                cnt_full = plsc.all_reduce_population_count(passed_c)
                rank_v = rank_v + jnp.where(found > 0, 0,
                                            jnp.where(has_k, cnt_here,
                                                      cnt_full))
                found = jnp.where(has_k, 1, found)
            plsc.store_scatter(gtc, [rank_v + 3 * M], one16,
                               mask=(iota == 0) & any_pass)
            return carry

        lax.fori_loop(0, n2, step, 0)

        tpa = jnp.zeros((L,), F32)
        for k in range(M // L):
            tpa = tpa + gtc[pl.ds(3 * M + k * L, L)]
        tp = jnp.broadcast_to(jnp.sum(tpa), (L,))
        g = ngv_v.astype(F32)
        npf = jnp.broadcast_to(n_p.astype(F32), (L,))
        has = n_p > 0
        prec = jnp.where(has, tp * _recip(jnp.maximum(npf, 1.0)), 0.0)
        rec = jnp.where(has, tp * _recip(jnp.maximum(g, 1.0)), 0.0)
        res_v[pl.ds(0, L)] = jnp.where(iota == b, prec, 0.0)
        pltpu.sync_copy(res_v, shared.at[pl.ds(b * L, L)])
        res_v[pl.ds(0, L)] = jnp.where(iota == b, rec, 0.0)
        pltpu.sync_copy(res_v, shared.at[pl.ds((B + b) * L, L)])

    plsc.subcore_barrier()

    @pl.when((c == 0) & (s == 0))
    def _reduce():
        pltpu.sync_copy(shared, redbuf)
        prec_vec = jnp.zeros((L,), F32)
        rec_vec = jnp.zeros((L,), F32)
        for i in range(B):
            prec_vec = prec_vec + redbuf[pl.ds(i * L, L)]
            rec_vec = rec_vec + redbuf[pl.ds((B + i) * L, L)]
        zero = jnp.zeros((L,), F32)
        ap1[pl.ds(0, L)] = zero
        ap1[pl.ds(L, L)] = zero
        ap2[pl.ds(0, L)] = zero
        ap2[pl.ds(L, L)] = zero
        plsc.store_scatter(ap1, [iota + 1], rec_vec, mask=iota < B)
        plsc.store_scatter(ap1, [jnp.full((L,), B + 1, I32)],
                           jnp.full((L,), 1.0, F32), mask=iota == 0)
        plsc.store_scatter(ap2, [iota + 1], prec_vec, mask=iota < B)
        mp = ap2[pl.ds(0, L)]
        mp = lax.rev(plsc.cummax(lax.rev(mp, (0,))), (0,))
        ap2[pl.ds(0, L)] = mp
        mrec = ap1[pl.ds(0, L)]
        mrec_n = plsc.load_gather(ap1, [iota + 1])
        mpre_n = plsc.load_gather(ap2, [iota + 1])
        terms = jnp.where(iota < B + 1, (mrec_n - mrec) * mpre_n, 0.0)
        apv = jnp.sum(terms)
        res_v[pl.ds(0, L)] = zero + apv
        pltpu.sync_copy(res_v, out_hbm)


def kernel(predicts, ground_truths, image_sizes):
    nb, n, _ = predicts.shape
    pT = jnp.transpose(predicts, (0, 2, 1)).astype(F32)
    pT = jnp.pad(pT, ((0, 0), (0, 0), (0, NPAD - n)), constant_values=-1.0)
    pred_arr = pT.reshape(nb, 6 * NPAD)
    gT = jnp.transpose(ground_truths, (0, 2, 1)).astype(F32)
    sz = jnp.pad(image_sizes.astype(F32), ((0, 0), (0, M - 2)))[:, None, :]
    gt_arr = jnp.concatenate([gT, sz], axis=1).reshape(nb, 6 * M)

    mesh = plsc.VectorSubcoreMesh(core_axis_name="c", subcore_axis_name="s")
    out = pl.kernel(
        _body,
        out_type=jax.ShapeDtypeStruct((L,), F32),
        mesh=mesh,
        compiler_params=pltpu.CompilerParams(needs_layout_passes=False),
        scratch_types=[
            pltpu.VMEM((6 * NPAD,), F32),
            pltpu.VMEM((6 * M,), F32),
            pltpu.VMEM((NPAD,), F32),
            pltpu.VMEM((NPAD,), F32),
            pltpu.VMEM((NPAD,), F32),
            pltpu.VMEM((NPAD,), F32),
            pltpu.VMEM((NPAD,), F32),
            pltpu.VMEM((NPAD,), F32),
            pltpu.VMEM((NPAD,), F32),
            pltpu.VMEM((NPAD,), F32),
            pltpu.VMEM((NPAD,), F32),
            pltpu.VMEM((4 * M,), F32),
            pltpu.VMEM((L,), F32),
            pltpu.VMEM((2 * B * L,), F32),
            pltpu.VMEM((2 * L,), F32),
            pltpu.VMEM((2 * L,), F32),
            pltpu.VMEM_SHARED((2 * B * L,), F32),
        ],
    )(pred_arr, gt_arr)
    return out[0]
